# baseline (speedup 1.0000x reference)
import functools

import jax
import jax.numpy as jnp
from jax import lax
from jax.experimental import pallas as pl
from jax.experimental.pallas import tpu as pltpu
from jax.experimental.pallas import tpu_sc as plsc

B, N, F = 16, 10000, 128
NC, NS = 2, 16
NW = NC * NS
WPB = NW // B
NV = F // 16

SC_N = 3200
SC_ROW0 = N - SC_N
ROWS = SC_N // WPB
CHUNK = 200
NCHUNK = ROWS // CHUNK
assert ROWS % CHUNK == 0 and CHUNK % 8 == 0 and SC_ROW0 % 8 == 0

_mesh = plsc.VectorSubcoreMesh(core_axis_name="c", subcore_axis_name="s")


@functools.partial(
    pl.kernel,
    out_type=jax.ShapeDtypeStruct((WPB * B * F,), jnp.float32),
    mesh=_mesh,
    scratch_types=[
        pltpu.VMEM((2, CHUNK, F), jnp.float32),
        pltpu.VMEM((F,), jnp.float32),
        pltpu.SemaphoreType.DMA,
        pltpu.SemaphoreType.DMA,
    ],
)
def _sc_pool(x_hbm, out_hbm, buf, accv, sem0, sem1):
    c = lax.axis_index("c")
    s = lax.axis_index("s")
    wid = c * NS + s
    b = wid // WPB
    h = wid % WPB
    row0 = SC_ROW0 + h * ROWS
    sems = (sem0, sem1)

    def start(i, slot):
        pltpu.make_async_copy(
            x_hbm.at[b, pl.ds(row0 + i * CHUNK, CHUNK), :],
            buf.at[slot], sems[slot]).start()

    def wait(slot):
        pltpu.make_async_copy(
            x_hbm.at[b, pl.ds(0, CHUNK), :],
            buf.at[slot], sems[slot]).wait()

    def consume(slot, acc):
        @plsc.parallel_loop(0, CHUNK, carry=acc, unroll=4)
        def row_body(r, a):
            a = list(a)
            for j in range(NV):
                a[j] = jnp.maximum(a[j], buf[slot, r, pl.ds(j * 16, 16)])
            return tuple(a)

        return row_body

    start(0, 0)
    start(1, 1)

    acc0 = tuple(jnp.full((16,), -jnp.inf, jnp.float32) for _ in range(NV))

    def pair_body(g, acc):
        for slot in range(2):
            i = 2 * g + slot
            wait(slot)
            acc = consume(slot, acc)
            nxt = i + 2

            @pl.when(nxt < NCHUNK)
            def _():
                start(nxt, slot)
        return acc

    acc = lax.fori_loop(0, NCHUNK // 2, pair_body, acc0)
    if NCHUNK % 2:
        wait(0)
        acc = consume(0, acc)

    for j in range(NV):
        accv[pl.ds(j * 16, 16)] = acc[j]
    pltpu.sync_copy(accv, out_hbm.at[pl.ds((h * B + b) * F, F)])


TC_BB = 2


def _tc_body(x_ref, o_ref):
    o_ref[...] = jnp.max(x_ref[...], axis=1, keepdims=True)


def _tc_pool(x, nrows):
    return pl.pallas_call(
        _tc_body,
        grid=(B // TC_BB,),
        in_specs=[pl.BlockSpec((TC_BB, nrows, F), lambda b: (b, 0, 0))],
        out_specs=pl.BlockSpec((TC_BB, 1, F), lambda b: (b, 0, 0)),
        out_shape=jax.ShapeDtypeStruct((B, 1, F), jnp.float32),
    )(x)


def kernel(x):
    sc = _sc_pool(x).reshape(WPB, B, F)
    tc = _tc_pool(x, N - SC_N).reshape(B, F)
    return jnp.maximum(tc, jnp.maximum(sc[0], sc[1]))


# speedup vs baseline: 1.0081x; 1.0081x over previous
import functools

import jax
import jax.numpy as jnp
from jax import lax
from jax.experimental import pallas as pl
from jax.experimental.pallas import tpu as pltpu
from jax.experimental.pallas import tpu_sc as plsc

B, N, F = 16, 10000, 128
NC, NS = 2, 16
NW = NC * NS
WPB = NW // B
NV = F // 16

SC_N = 1600
SC_ROW0 = N - SC_N
ROWS = SC_N // WPB
CHUNK = 200
NCHUNK = ROWS // CHUNK
assert ROWS % CHUNK == 0 and CHUNK % 8 == 0 and SC_ROW0 % 8 == 0

_mesh = plsc.VectorSubcoreMesh(core_axis_name="c", subcore_axis_name="s")


@functools.partial(
    pl.kernel,
    out_type=jax.ShapeDtypeStruct((WPB * B * F,), jnp.float32),
    mesh=_mesh,
    scratch_types=[
        pltpu.VMEM((2, CHUNK, F), jnp.float32),
        pltpu.VMEM((F,), jnp.float32),
        pltpu.SemaphoreType.DMA,
        pltpu.SemaphoreType.DMA,
    ],
)
def _sc_pool(x_hbm, out_hbm, buf, accv, sem0, sem1):
    c = lax.axis_index("c")
    s = lax.axis_index("s")
    wid = c * NS + s
    b = wid // WPB
    h = wid % WPB
    row0 = SC_ROW0 + h * ROWS
    sems = (sem0, sem1)

    def start(i, slot):
        pltpu.make_async_copy(
            x_hbm.at[b, pl.ds(row0 + i * CHUNK, CHUNK), :],
            buf.at[slot], sems[slot]).start()

    def wait(slot):
        pltpu.make_async_copy(
            x_hbm.at[b, pl.ds(0, CHUNK), :],
            buf.at[slot], sems[slot]).wait()

    def consume(slot, acc):
        @plsc.parallel_loop(0, CHUNK, carry=acc, unroll=4)
        def row_body(r, a):
            a = list(a)
            for j in range(NV):
                a[j] = jnp.maximum(a[j], buf[slot, r, pl.ds(j * 16, 16)])
            return tuple(a)

        return row_body

    start(0, 0)
    start(1, 1)

    acc0 = tuple(jnp.full((16,), -jnp.inf, jnp.float32) for _ in range(NV))

    def pair_body(g, acc):
        for slot in range(2):
            i = 2 * g + slot
            wait(slot)
            acc = consume(slot, acc)
            nxt = i + 2

            @pl.when(nxt < NCHUNK)
            def _():
                start(nxt, slot)
        return acc

    acc = lax.fori_loop(0, NCHUNK // 2, pair_body, acc0)
    if NCHUNK % 2:
        wait(0)
        acc = consume(0, acc)

    for j in range(NV):
        accv[pl.ds(j * 16, 16)] = acc[j]
    pltpu.sync_copy(accv, out_hbm.at[pl.ds((h * B + b) * F, F)])


TC_BB = 2


def _tc_body(x_ref, o_ref):
    o_ref[...] = jnp.max(x_ref[...], axis=1, keepdims=True)


def _tc_pool(x, nrows):
    return pl.pallas_call(
        _tc_body,
        grid=(B // TC_BB,),
        in_specs=[pl.BlockSpec((TC_BB, nrows, F), lambda b: (b, 0, 0))],
        out_specs=pl.BlockSpec((TC_BB, 1, F), lambda b: (b, 0, 0)),
        out_shape=jax.ShapeDtypeStruct((B, 1, F), jnp.float32),
    )(x)


def kernel(x):
    sc = _sc_pool(x).reshape(WPB, B, F)
    tc = _tc_pool(x, N - SC_N).reshape(B, F)
    return jnp.maximum(tc, jnp.maximum(sc[0], sc[1]))
